# Initial kernel scaffold; baseline (speedup 1.0000x reference)
import jax
import jax.numpy as jnp
from jax.experimental import pallas as pl


def kernel(image, W):
    raise NotImplementedError("write your pallas kernel here")


# speedup vs baseline: 1.0992x; 1.0992x over previous
import functools

import jax
import jax.numpy as jnp
from jax import lax
from jax.experimental import pallas as pl
from jax.experimental.pallas import tpu as pltpu
from jax.experimental.pallas import tpu_sc as plsc

_NC, _NS = 2, 16
_NW = _NC * _NS
_GSZ = 128


@functools.lru_cache(maxsize=None)
def _make_gather(n_rows: int, feat: int):
    assert n_rows % (_NW * _GSZ) == 0
    n_groups = n_rows // _GSZ
    groups_per_w = n_groups // _NW
    mesh = plsc.VectorSubcoreMesh(core_axis_name="c", subcore_axis_name="s")

    @functools.partial(
        pl.kernel,
        out_type=jax.ShapeDtypeStruct((n_rows, feat), jnp.float32),
        mesh=mesh,
        scratch_types=[
            pltpu.VMEM((groups_per_w, _GSZ), jnp.int32),
            pltpu.VMEM((_GSZ, feat), jnp.float32),
            pltpu.SemaphoreType.DMA,
        ],
        compiler_params=pltpu.CompilerParams(use_tc_tiling_on_sc=False),
    )
    def gather_kernel(table_hbm, idx_hbm, out_hbm, idx_v, rows_v, sem):
        wid = lax.axis_index("s") * _NC + lax.axis_index("c")
        base_g = wid * groups_per_w
        pltpu.sync_copy(idx_hbm.at[pl.ds(base_g, groups_per_w)], idx_v)

        @pl.loop(0, groups_per_w)
        def _(j):
            pltpu.async_copy(table_hbm.at[idx_v.at[j]], rows_v, sem).wait()
            pltpu.sync_copy(rows_v, out_hbm.at[pl.ds((base_g + j) * _GSZ, _GSZ)])

    return gather_kernel


def kernel(image, W):
    B, A = image.shape
    V, F = W.shape
    idx = image.reshape(-1).astype(jnp.int32)
    n_rows = B * A
    idx2 = idx.reshape(n_rows // _GSZ, _GSZ)
    rows = _make_gather(n_rows, F)(W, idx2)
    return rows.reshape(B, A * F)


# speedup vs baseline: 1.1941x; 1.0863x over previous
import functools

import jax
import jax.numpy as jnp
from jax import lax
from jax.experimental import pallas as pl
from jax.experimental.pallas import tpu as pltpu
from jax.experimental.pallas import tpu_sc as plsc

_NC, _NS = 2, 16
_NW = _NC * _NS
_GSZ = 128


@functools.lru_cache(maxsize=None)
def _make_gather(n_rows: int, feat: int):
    assert n_rows % (_NW * _GSZ) == 0
    n_groups = n_rows // _GSZ
    groups_per_w = n_groups // _NW
    mesh = plsc.VectorSubcoreMesh(core_axis_name="c", subcore_axis_name="s")

    K = 4
    S = groups_per_w // K
    assert groups_per_w % K == 0 and S % 2 == 0

    @functools.partial(
        pl.kernel,
        out_type=jax.ShapeDtypeStruct((n_rows, feat), jnp.float32),
        mesh=mesh,
        scratch_types=[
            pltpu.VMEM((groups_per_w, _GSZ), jnp.int32),
            pltpu.VMEM((K * _GSZ, feat), jnp.float32),
            pltpu.VMEM((K * _GSZ, feat), jnp.float32),
            pltpu.SemaphoreType.DMA,
            pltpu.SemaphoreType.DMA,
            pltpu.SemaphoreType.DMA,
        ],
        compiler_params=pltpu.CompilerParams(use_tc_tiling_on_sc=False),
    )
    def gather_kernel(table_hbm, idx_hbm, out_hbm, idx_v, rows_a, rows_b,
                      gsem, wsem_a, wsem_b):
        wid = lax.axis_index("s") * _NC + lax.axis_index("c")
        base_g = wid * groups_per_w
        pltpu.sync_copy(idx_hbm.at[pl.ds(base_g, groups_per_w)], idx_v)

        def fire_gathers(buf, ss):
            for t in range(K):
                pltpu.async_copy(table_hbm.at[idx_v.at[ss * K + t]],
                                 buf.at[pl.ds(t * _GSZ, _GSZ)], gsem)

        def out_slice(ss):
            return out_hbm.at[pl.ds((base_g + ss * K) * _GSZ, K * _GSZ)]

        fire_gathers(rows_a, 0)

        @pl.loop(0, S, step=2)
        def _(s):
            for cur, nxt, wsem_cur, wsem_nxt, off in (
                    (rows_a, rows_b, wsem_a, wsem_b, 0),
                    (rows_b, rows_a, wsem_b, wsem_a, 1)):
                ss = s + off
                for t in range(K):
                    pltpu.make_async_copy(
                        table_hbm.at[idx_v.at[ss * K + t]],
                        cur.at[pl.ds(t * _GSZ, _GSZ)], gsem).wait()
                @pl.when(ss + 1 < S)
                def _():
                    @pl.when(ss >= 1)
                    def _():
                        pltpu.make_async_copy(nxt, out_slice(ss - 1),
                                              wsem_nxt).wait()
                    fire_gathers(nxt, ss + 1)
                pltpu.async_copy(cur, out_slice(ss), wsem_cur)

        pltpu.make_async_copy(rows_a, out_slice(S - 2), wsem_a).wait()
        pltpu.make_async_copy(rows_b, out_slice(S - 1), wsem_b).wait()

    return gather_kernel


def kernel(image, W):
    B, A = image.shape
    V, F = W.shape
    idx = image.reshape(-1).astype(jnp.int32)
    n_rows = B * A
    idx2 = idx.reshape(n_rows // _GSZ, _GSZ)
    rows = _make_gather(n_rows, F)(W, idx2)
    return rows.reshape(B, A * F)
